# baseline (speedup 1.0000x reference)
import functools

import jax
import jax.numpy as jnp
from jax import lax
from jax.experimental import pallas as pl
from jax.experimental.pallas import tpu as pltpu
from jax.experimental.pallas import tpu_sc as plsc

N = 10000
NPAD = 10240
HID = 128
E = 320000
B = 64
NC = 2
NS = 16
NW = NC * NS
CL = 128
CHUNKS = 80
G = 8
NGRP = CHUNKS // G
EPAD = NW * CHUNKS * CL
NBUF = 2
RPT = NPAD // NS
CNTW = 16

_mesh = plsc.VectorSubcoreMesh(core_axis_name="c", subcore_axis_name="s")


GSTG = 16
NGST = CHUNKS // GSTG


def _seg_body(h_hbm, gidx_hbm, sidx_hbm, out_hbm, gidx_v, sidx_v, rows, acc,
              g0, g1, s0, s1, isem):
    gsems = (g0, g1)
    ssems = (s0, s1)
    cid = lax.axis_index("c")
    sid = lax.axis_index("s")
    wid = sid * NC + cid

    def _zr(r, carry):
        for u in range(HID // 16):
            rows[0, r, pl.ds(u * 16, 16)] = jnp.zeros((16,), jnp.float32)
        return carry
    lax.fori_loop(0, CL, _zr, 0)
    base = sid * RPT
    for k in range(RPT // CL):
        pltpu.sync_copy(rows.at[0], acc.at[pl.ds(base + k * CL, CL)])
    plsc.subcore_barrier()

    pltpu.sync_copy(gidx_hbm.at[wid, pl.ds(0, GSTG)], gidx_v.at[0])
    pltpu.sync_copy(sidx_hbm.at[wid, pl.ds(0, GSTG)], sidx_v.at[0])

    def _grp(t, carry):
        islot = lax.rem(t, 2)

        @pl.when(t < NGST - 1)
        def _():
            nxt = (t + 1) * GSTG
            pltpu.async_copy(gidx_hbm.at[wid, pl.ds(nxt, GSTG)],
                             gidx_v.at[1 - islot], isem)
            pltpu.async_copy(sidx_hbm.at[wid, pl.ds(nxt, GSTG)],
                             sidx_v.at[1 - islot], isem)

        for b in range(2):
            pltpu.async_copy(h_hbm.at[gidx_v.at[islot, b]], rows.at[b],
                             gsems[b])
        for kk in range(0, GSTG, 2):
            for b in range(2):
                pltpu.make_async_copy(h_hbm.at[gidx_v.at[islot, kk + b]],
                                      rows.at[b], gsems[b]).wait()
            for b in range(2):
                pltpu.async_copy(rows.at[b], acc.at[sidx_v.at[islot, kk + b]],
                                 ssems[b], add=True)
            for b in range(2):
                pltpu.make_async_copy(rows.at[b],
                                      acc.at[sidx_v.at[islot, kk + b]],
                                      ssems[b]).wait()
            if kk + 2 < GSTG:
                for b in range(2):
                    pltpu.async_copy(h_hbm.at[gidx_v.at[islot, kk + 2 + b]],
                                     rows.at[b], gsems[b])

        @pl.when(t < NGST - 1)
        def _():
            pltpu.make_async_copy(gidx_hbm.at[wid, pl.ds(0, GSTG)],
                                  gidx_v.at[1 - islot], isem).wait()
            pltpu.make_async_copy(sidx_hbm.at[wid, pl.ds(0, GSTG)],
                                  sidx_v.at[1 - islot], isem).wait()
        return carry
    lax.fori_loop(0, NGST, _grp, 0)

    plsc.subcore_barrier()
    pltpu.sync_copy(acc.at[pl.ds(base, RPT)], out_hbm.at[cid, pl.ds(base, RPT)])


_segsum = functools.partial(
    pl.kernel,
    out_type=jax.ShapeDtypeStruct((NC, NPAD, HID), jnp.float32),
    mesh=_mesh,
    scratch_types=[
        pltpu.VMEM((2, GSTG, CL), jnp.int32),
        pltpu.VMEM((2, GSTG, CL), jnp.int32),
        pltpu.VMEM((2, CL, HID), jnp.float32),
        pltpu.VMEM_SHARED((NPAD, HID), jnp.float32),
        pltpu.SemaphoreType.DMA,
        pltpu.SemaphoreType.DMA,
        pltpu.SemaphoreType.DMA,
        pltpu.SemaphoreType.DMA,
        pltpu.SemaphoreType.DMA,
    ],
)(_seg_body)


def _cnt_body(sidx_hbm, didx_hbm, const_hbm, out_hbm, sidx_v, didx_v, ones_v,
              acc, sem):
    cid = lax.axis_index("c")
    sid = lax.axis_index("s")
    wid = sid * NC + cid
    pltpu.sync_copy(sidx_hbm.at[wid], sidx_v)
    pltpu.sync_copy(didx_hbm.at[wid], didx_v)

    pltpu.sync_copy(const_hbm.at[0], ones_v)
    base = sid * RPT
    for k in range(RPT // CL):
        pltpu.sync_copy(ones_v, acc.at[pl.ds(base + k * CL, CL)])
    plsc.subcore_barrier()

    def _pass(idx_v, plane):
        pltpu.sync_copy(const_hbm.at[plane], ones_v)

        FG = 4

        def _grp(g, carry):
            for b in range(FG):
                pltpu.async_copy(ones_v, acc.at[idx_v.at[g * FG + b]], sem,
                                 add=True)
            for b in range(FG):
                pltpu.make_async_copy(ones_v, acc.at[idx_v.at[g * FG + b]],
                                      sem).wait()
            return carry
        lax.fori_loop(0, CHUNKS // FG, _grp, 0)

    _pass(sidx_v, 1)
    _pass(didx_v, 2)

    plsc.subcore_barrier()
    pltpu.sync_copy(acc.at[pl.ds(base, RPT)], out_hbm.at[cid, pl.ds(base, RPT)])


_counts = functools.partial(
    pl.kernel,
    out_type=jax.ShapeDtypeStruct((NC, NPAD, HID), jnp.float32),
    mesh=_mesh,
    scratch_types=[
        pltpu.VMEM((CHUNKS, CL), jnp.int32),
        pltpu.VMEM((CHUNKS, CL), jnp.int32),
        pltpu.VMEM((CL, HID), jnp.float32),
        pltpu.VMEM_SHARED((NPAD, HID), jnp.float32),
        pltpu.SemaphoreType.DMA,
    ],
)(_cnt_body)


def _enc_body(x_ref, w1_ref, b1_ref, w2_ref, b2_ref, o_ref):
    h = jnp.maximum(
        jnp.dot(x_ref[...], w1_ref[...], preferred_element_type=jnp.float32)
        + b1_ref[...], 0.0)
    o_ref[...] = jnp.maximum(
        jnp.dot(h, w2_ref[...], preferred_element_type=jnp.float32)
        + b2_ref[...], 0.0)


_ENC_BLK = 1024
_enc = pl.pallas_call(
    _enc_body,
    grid=(NPAD // _ENC_BLK,),
    in_specs=[
        pl.BlockSpec((_ENC_BLK, HID), lambda i: (i, 0)),
        pl.BlockSpec((HID, HID), lambda i: (0, 0)),
        pl.BlockSpec((1, HID), lambda i: (0, 0)),
        pl.BlockSpec((HID, HID), lambda i: (0, 0)),
        pl.BlockSpec((1, HID), lambda i: (0, 0)),
    ],
    out_specs=pl.BlockSpec((_ENC_BLK, HID), lambda i: (i, 0)),
    out_shape=jax.ShapeDtypeStruct((NPAD, HID), jnp.float32),
)


def _upd_body(col, h_ref, p_ref, c_ref, w_ref, b_ref, o_ref):
    s = p_ref[0] + p_ref[1]
    cnt = c_ref[0, :, col:col + 1] + c_ref[1, :, col:col + 1]
    m = s / jnp.maximum(cnt, 1.0)
    o_ref[...] = jnp.maximum(
        h_ref[...]
        + jnp.dot(m, w_ref[...], preferred_element_type=jnp.float32)
        + b_ref[...], 0.0)


_UPD_BLK = 1280


def _make_upd(col):
    return pl.pallas_call(
        functools.partial(_upd_body, col),
        grid=(NPAD // _UPD_BLK,),
        in_specs=[
            pl.BlockSpec((_UPD_BLK, HID), lambda i: (i, 0)),
            pl.BlockSpec((NC, _UPD_BLK, HID), lambda i: (0, i, 0)),
            pl.BlockSpec((NC, _UPD_BLK, HID), lambda i: (0, i, 0)),
            pl.BlockSpec((HID, HID), lambda i: (0, 0)),
            pl.BlockSpec((1, HID), lambda i: (0, 0)),
        ],
        out_specs=pl.BlockSpec((_UPD_BLK, HID), lambda i: (i, 0)),
        out_shape=jax.ShapeDtypeStruct((NPAD, HID), jnp.float32),
    )


_upd_s = _make_upd(0)
_upd_d = _make_upd(1)


_POOL_BLK = 1000
_GP = N // _POOL_BLK


def _pool_body(hv_ref, hc_ref, bv_ref, bc_ref, w1_ref, b1_ref, w2_ref, b2_ref,
               o_ref, pv_s, pc_s, cv_s, cc_s):
    i = pl.program_id(0)

    @pl.when(i == 0)
    def _():
        pv_s[...] = jnp.zeros((B, HID), jnp.float32)
        pc_s[...] = jnp.zeros((B, HID), jnp.float32)
        cv_s[...] = jnp.zeros((B, HID), jnp.float32)
        cc_s[...] = jnp.zeros((B, HID), jnp.float32)

    ids = lax.broadcasted_iota(jnp.int32, (B, _POOL_BLK), 0)
    ohv = (ids == bv_ref[0, 0, :][None, :]).astype(jnp.float32)
    ohc = (ids == bc_ref[0, 0, :][None, :]).astype(jnp.float32)
    pv_s[...] += jnp.dot(ohv, hv_ref[...], preferred_element_type=jnp.float32)
    pc_s[...] += jnp.dot(ohc, hc_ref[...], preferred_element_type=jnp.float32)
    cv_s[...] += jnp.broadcast_to(jnp.sum(ohv, axis=1)[:, None], (B, HID))
    cc_s[...] += jnp.broadcast_to(jnp.sum(ohc, axis=1)[:, None], (B, HID))

    @pl.when(i == _GP - 1)
    def _():
        mv = pv_s[...] / jnp.maximum(cv_s[...], 1.0)
        mc = pc_s[...] / jnp.maximum(cc_s[...], 1.0)
        emb = jnp.concatenate([mv, mc], axis=1)
        hh = jnp.maximum(
            jnp.dot(emb, w1_ref[...], preferred_element_type=jnp.float32)
            + b1_ref[...], 0.0)
        o_ref[...] = (jnp.dot(hh, w2_ref[...],
                              preferred_element_type=jnp.float32)
                      + b2_ref[...])


_pool = pl.pallas_call(
    _pool_body,
    grid=(_GP,),
    in_specs=[
        pl.BlockSpec((_POOL_BLK, HID), lambda i: (i, 0)),
        pl.BlockSpec((_POOL_BLK, HID), lambda i: (i, 0)),
        pl.BlockSpec((1, 1, _POOL_BLK), lambda i: (i, 0, 0)),
        pl.BlockSpec((1, 1, _POOL_BLK), lambda i: (i, 0, 0)),
        pl.BlockSpec((2 * HID, 4 * HID), lambda i: (0, 0)),
        pl.BlockSpec((1, 4 * HID), lambda i: (0, 0)),
        pl.BlockSpec((4 * HID, HID), lambda i: (0, 0)),
        pl.BlockSpec((1, HID), lambda i: (0, 0)),
    ],
    out_specs=pl.BlockSpec((B, HID), lambda i: (0, 0)),
    out_shape=jax.ShapeDtypeStruct((B, HID), jnp.float32),
    scratch_shapes=[
        pltpu.VMEM((B, HID), jnp.float32),
        pltpu.VMEM((B, HID), jnp.float32),
        pltpu.VMEM((B, HID), jnp.float32),
        pltpu.VMEM((B, HID), jnp.float32),
    ],
)


def kernel(x_vals, x_cons, edge_index, batch_vals, batch_cons,
           W_ev1, b_ev1, W_ev2, b_ev2, W_ec1, b_ec1, W_ec2, b_ec2,
           W_v2c_0, b_v2c_0, W_c2v_0, b_c2v_0, W_v2c_1, b_v2c_1,
           W_c2v_1, b_c2v_1, W_p1, b_p1, W_p2, b_p2):
    src = edge_index[0].astype(jnp.int32)
    dst = edge_index[1].astype(jnp.int32)
    pad = jnp.full((EPAD - E,), N, dtype=jnp.int32)
    src_p = jnp.concatenate([src, pad]).reshape(NW, CHUNKS, CL)
    dst_p = jnp.concatenate([dst, pad]).reshape(NW, CHUNKS, CL)
    xv = jnp.pad(x_vals, ((0, NPAD - N), (0, 0)))
    xc = jnp.pad(x_cons, ((0, NPAD - N), (0, 0)))

    onehot_rows = (jnp.zeros((3, CL, HID), jnp.float32)
                   .at[1, :, 0].set(1.0).at[2, :, 1].set(1.0))
    cnt = _counts(src_p, dst_p, onehot_rows)
    hv = _enc(xv, W_ev1, b_ev1[None], W_ev2, b_ev2[None])
    hc = _enc(xc, W_ec1, b_ec1[None], W_ec2, b_ec2[None])

    for Wvc, bvc, Wcv, bcv in ((W_v2c_0, b_v2c_0, W_c2v_0, b_c2v_0),
                               (W_v2c_1, b_v2c_1, W_c2v_1, b_c2v_1)):
        ps = _segsum(hv, src_p, dst_p)
        hc = _upd_d(hc, ps, cnt, Wvc, bvc[None])
        pv = _segsum(hc, dst_p, src_p)
        hv = _upd_s(hv, pv, cnt, Wcv, bcv[None])

    bv3 = batch_vals.astype(jnp.int32).reshape(_GP, 1, _POOL_BLK)
    bc3 = batch_cons.astype(jnp.int32).reshape(_GP, 1, _POOL_BLK)
    return _pool(hv, hc, bv3, bc3, W_p1, b_p1[None], W_p2, b_p2[None])


# speedup vs baseline: 1.1002x; 1.1002x over previous
import functools

import jax
import jax.numpy as jnp
from jax import lax
from jax.experimental import pallas as pl
from jax.experimental.pallas import tpu as pltpu
from jax.experimental.pallas import tpu_sc as plsc

N = 10000
NPAD = 10240
HID = 128
E = 320000
B = 64
NC = 2
NS = 16
NW = NC * NS
CL = 128
CHUNKS = 80
G = 8
NGRP = CHUNKS // G
EPAD = NW * CHUNKS * CL
NBUF = 2
RPT = NPAD // NS
CNTW = 16

_mesh = plsc.VectorSubcoreMesh(core_axis_name="c", subcore_axis_name="s")


GSTG = 16
NGST = CHUNKS // GSTG


def _seg_body(h_hbm, gidx_hbm, sidx_hbm, out_hbm, gidx_v, sidx_v, rows, acc,
              g0, g1, s0, s1, isem):
    gsems = (g0, g1)
    ssems = (s0, s1)
    cid = lax.axis_index("c")
    sid = lax.axis_index("s")
    wid = sid * NC + cid

    def _zr(r, carry):
        for u in range(HID // 16):
            rows[0, r, pl.ds(u * 16, 16)] = jnp.zeros((16,), jnp.float32)
        return carry
    lax.fori_loop(0, CL, _zr, 0)
    base = sid * RPT
    for k in range(RPT // CL):
        pltpu.sync_copy(rows.at[0], acc.at[pl.ds(base + k * CL, CL)])
    plsc.subcore_barrier()

    pltpu.sync_copy(gidx_hbm.at[wid, pl.ds(0, GSTG)], gidx_v.at[0])
    pltpu.sync_copy(sidx_hbm.at[wid, pl.ds(0, GSTG)], sidx_v.at[0])
    for b in range(2):
        pltpu.async_copy(h_hbm.at[gidx_v.at[0, b]], rows.at[b], gsems[b])

    def _grp(t, carry):
        islot = lax.rem(t, 2)

        @pl.when(t < NGST - 1)
        def _():
            nxt = (t + 1) * GSTG
            pltpu.async_copy(gidx_hbm.at[wid, pl.ds(nxt, GSTG)],
                             gidx_v.at[1 - islot], isem)
            pltpu.async_copy(sidx_hbm.at[wid, pl.ds(nxt, GSTG)],
                             sidx_v.at[1 - islot], isem)

        for k in range(GSTG):
            b = k % 2
            pltpu.make_async_copy(h_hbm.at[gidx_v.at[islot, k]], rows.at[b],
                                  gsems[b]).wait()
            pltpu.async_copy(rows.at[b], acc.at[sidx_v.at[islot, k]],
                             ssems[b], add=True)
            pltpu.make_async_copy(rows.at[b], acc.at[sidx_v.at[islot, k]],
                                  ssems[b]).wait()
            if k + 2 < GSTG:
                pltpu.async_copy(h_hbm.at[gidx_v.at[islot, k + 2]],
                                 rows.at[b], gsems[b])
            else:
                @pl.when(t < NGST - 1)
                def _(k=k, b=b):
                    if k == GSTG - 2:
                        pltpu.make_async_copy(
                            gidx_hbm.at[wid, pl.ds(0, GSTG)],
                            gidx_v.at[1 - islot], isem).wait()
                        pltpu.make_async_copy(
                            sidx_hbm.at[wid, pl.ds(0, GSTG)],
                            sidx_v.at[1 - islot], isem).wait()
                    pltpu.async_copy(
                        h_hbm.at[gidx_v.at[1 - islot, k + 2 - GSTG]],
                        rows.at[b], gsems[b])
        return carry
    lax.fori_loop(0, NGST, _grp, 0)

    plsc.subcore_barrier()
    pltpu.sync_copy(acc.at[pl.ds(base, RPT)], out_hbm.at[cid, pl.ds(base, RPT)])


_segsum = functools.partial(
    pl.kernel,
    out_type=jax.ShapeDtypeStruct((NC, NPAD, HID), jnp.float32),
    mesh=_mesh,
    scratch_types=[
        pltpu.VMEM((2, GSTG, CL), jnp.int32),
        pltpu.VMEM((2, GSTG, CL), jnp.int32),
        pltpu.VMEM((2, CL, HID), jnp.float32),
        pltpu.VMEM_SHARED((NPAD, HID), jnp.float32),
        pltpu.SemaphoreType.DMA,
        pltpu.SemaphoreType.DMA,
        pltpu.SemaphoreType.DMA,
        pltpu.SemaphoreType.DMA,
        pltpu.SemaphoreType.DMA,
    ],
)(_seg_body)


def _cnt_body(sidx_hbm, didx_hbm, const_hbm, out_hbm, sidx_v, didx_v, ones_v,
              acc, sem):
    cid = lax.axis_index("c")
    sid = lax.axis_index("s")
    wid = sid * NC + cid
    pltpu.sync_copy(sidx_hbm.at[wid], sidx_v)
    pltpu.sync_copy(didx_hbm.at[wid], didx_v)

    pltpu.sync_copy(const_hbm.at[0], ones_v)
    base = sid * RPT
    for k in range(RPT // CL):
        pltpu.sync_copy(ones_v, acc.at[pl.ds(base + k * CL, CL)])
    plsc.subcore_barrier()

    def _pass(idx_v, plane):
        pltpu.sync_copy(const_hbm.at[plane], ones_v)

        FG = 4

        def _grp(g, carry):
            for b in range(FG):
                pltpu.async_copy(ones_v, acc.at[idx_v.at[g * FG + b]], sem,
                                 add=True)
            for b in range(FG):
                pltpu.make_async_copy(ones_v, acc.at[idx_v.at[g * FG + b]],
                                      sem).wait()
            return carry
        lax.fori_loop(0, CHUNKS // FG, _grp, 0)

    _pass(sidx_v, 1)
    _pass(didx_v, 2)

    plsc.subcore_barrier()
    pltpu.sync_copy(acc.at[pl.ds(base, RPT)], out_hbm.at[cid, pl.ds(base, RPT)])


_counts = functools.partial(
    pl.kernel,
    out_type=jax.ShapeDtypeStruct((NC, NPAD, HID), jnp.float32),
    mesh=_mesh,
    scratch_types=[
        pltpu.VMEM((CHUNKS, CL), jnp.int32),
        pltpu.VMEM((CHUNKS, CL), jnp.int32),
        pltpu.VMEM((CL, HID), jnp.float32),
        pltpu.VMEM_SHARED((NPAD, HID), jnp.float32),
        pltpu.SemaphoreType.DMA,
    ],
)(_cnt_body)


def _enc_body(x_ref, w1_ref, b1_ref, w2_ref, b2_ref, o_ref):
    h = jnp.maximum(
        jnp.dot(x_ref[...], w1_ref[...], preferred_element_type=jnp.float32)
        + b1_ref[...], 0.0)
    o_ref[...] = jnp.maximum(
        jnp.dot(h, w2_ref[...], preferred_element_type=jnp.float32)
        + b2_ref[...], 0.0)


_ENC_BLK = 1024
_enc = pl.pallas_call(
    _enc_body,
    grid=(NPAD // _ENC_BLK,),
    in_specs=[
        pl.BlockSpec((_ENC_BLK, HID), lambda i: (i, 0)),
        pl.BlockSpec((HID, HID), lambda i: (0, 0)),
        pl.BlockSpec((1, HID), lambda i: (0, 0)),
        pl.BlockSpec((HID, HID), lambda i: (0, 0)),
        pl.BlockSpec((1, HID), lambda i: (0, 0)),
    ],
    out_specs=pl.BlockSpec((_ENC_BLK, HID), lambda i: (i, 0)),
    out_shape=jax.ShapeDtypeStruct((NPAD, HID), jnp.float32),
)


def _upd_body(col, h_ref, p_ref, c_ref, w_ref, b_ref, o_ref):
    s = p_ref[0] + p_ref[1]
    cnt = c_ref[0, :, col:col + 1] + c_ref[1, :, col:col + 1]
    m = s / jnp.maximum(cnt, 1.0)
    o_ref[...] = jnp.maximum(
        h_ref[...]
        + jnp.dot(m, w_ref[...], preferred_element_type=jnp.float32)
        + b_ref[...], 0.0)


_UPD_BLK = 1280


def _make_upd(col):
    return pl.pallas_call(
        functools.partial(_upd_body, col),
        grid=(NPAD // _UPD_BLK,),
        in_specs=[
            pl.BlockSpec((_UPD_BLK, HID), lambda i: (i, 0)),
            pl.BlockSpec((NC, _UPD_BLK, HID), lambda i: (0, i, 0)),
            pl.BlockSpec((NC, _UPD_BLK, HID), lambda i: (0, i, 0)),
            pl.BlockSpec((HID, HID), lambda i: (0, 0)),
            pl.BlockSpec((1, HID), lambda i: (0, 0)),
        ],
        out_specs=pl.BlockSpec((_UPD_BLK, HID), lambda i: (i, 0)),
        out_shape=jax.ShapeDtypeStruct((NPAD, HID), jnp.float32),
    )


_upd_s = _make_upd(0)
_upd_d = _make_upd(1)


_POOL_BLK = 1000
_GP = N // _POOL_BLK


def _pool_body(hv_ref, hc_ref, bv_ref, bc_ref, w1_ref, b1_ref, w2_ref, b2_ref,
               o_ref, pv_s, pc_s, cv_s, cc_s):
    i = pl.program_id(0)

    @pl.when(i == 0)
    def _():
        pv_s[...] = jnp.zeros((B, HID), jnp.float32)
        pc_s[...] = jnp.zeros((B, HID), jnp.float32)
        cv_s[...] = jnp.zeros((B, HID), jnp.float32)
        cc_s[...] = jnp.zeros((B, HID), jnp.float32)

    ids = lax.broadcasted_iota(jnp.int32, (B, _POOL_BLK), 0)
    ohv = (ids == bv_ref[0, 0, :][None, :]).astype(jnp.float32)
    ohc = (ids == bc_ref[0, 0, :][None, :]).astype(jnp.float32)
    pv_s[...] += jnp.dot(ohv, hv_ref[...], preferred_element_type=jnp.float32)
    pc_s[...] += jnp.dot(ohc, hc_ref[...], preferred_element_type=jnp.float32)
    cv_s[...] += jnp.broadcast_to(jnp.sum(ohv, axis=1)[:, None], (B, HID))
    cc_s[...] += jnp.broadcast_to(jnp.sum(ohc, axis=1)[:, None], (B, HID))

    @pl.when(i == _GP - 1)
    def _():
        mv = pv_s[...] / jnp.maximum(cv_s[...], 1.0)
        mc = pc_s[...] / jnp.maximum(cc_s[...], 1.0)
        emb = jnp.concatenate([mv, mc], axis=1)
        hh = jnp.maximum(
            jnp.dot(emb, w1_ref[...], preferred_element_type=jnp.float32)
            + b1_ref[...], 0.0)
        o_ref[...] = (jnp.dot(hh, w2_ref[...],
                              preferred_element_type=jnp.float32)
                      + b2_ref[...])


_pool = pl.pallas_call(
    _pool_body,
    grid=(_GP,),
    in_specs=[
        pl.BlockSpec((_POOL_BLK, HID), lambda i: (i, 0)),
        pl.BlockSpec((_POOL_BLK, HID), lambda i: (i, 0)),
        pl.BlockSpec((1, 1, _POOL_BLK), lambda i: (i, 0, 0)),
        pl.BlockSpec((1, 1, _POOL_BLK), lambda i: (i, 0, 0)),
        pl.BlockSpec((2 * HID, 4 * HID), lambda i: (0, 0)),
        pl.BlockSpec((1, 4 * HID), lambda i: (0, 0)),
        pl.BlockSpec((4 * HID, HID), lambda i: (0, 0)),
        pl.BlockSpec((1, HID), lambda i: (0, 0)),
    ],
    out_specs=pl.BlockSpec((B, HID), lambda i: (0, 0)),
    out_shape=jax.ShapeDtypeStruct((B, HID), jnp.float32),
    scratch_shapes=[
        pltpu.VMEM((B, HID), jnp.float32),
        pltpu.VMEM((B, HID), jnp.float32),
        pltpu.VMEM((B, HID), jnp.float32),
        pltpu.VMEM((B, HID), jnp.float32),
    ],
)


def kernel(x_vals, x_cons, edge_index, batch_vals, batch_cons,
           W_ev1, b_ev1, W_ev2, b_ev2, W_ec1, b_ec1, W_ec2, b_ec2,
           W_v2c_0, b_v2c_0, W_c2v_0, b_c2v_0, W_v2c_1, b_v2c_1,
           W_c2v_1, b_c2v_1, W_p1, b_p1, W_p2, b_p2):
    src = edge_index[0].astype(jnp.int32)
    dst = edge_index[1].astype(jnp.int32)
    pad = jnp.full((EPAD - E,), N, dtype=jnp.int32)
    src_p = jnp.concatenate([src, pad]).reshape(NW, CHUNKS, CL)
    dst_p = jnp.concatenate([dst, pad]).reshape(NW, CHUNKS, CL)
    xv = jnp.pad(x_vals, ((0, NPAD - N), (0, 0)))
    xc = jnp.pad(x_cons, ((0, NPAD - N), (0, 0)))

    onehot_rows = (jnp.zeros((3, CL, HID), jnp.float32)
                   .at[1, :, 0].set(1.0).at[2, :, 1].set(1.0))
    cnt = _counts(src_p, dst_p, onehot_rows)
    hv = _enc(xv, W_ev1, b_ev1[None], W_ev2, b_ev2[None])
    hc = _enc(xc, W_ec1, b_ec1[None], W_ec2, b_ec2[None])

    for Wvc, bvc, Wcv, bcv in ((W_v2c_0, b_v2c_0, W_c2v_0, b_c2v_0),
                               (W_v2c_1, b_v2c_1, W_c2v_1, b_c2v_1)):
        ps = _segsum(hv, src_p, dst_p)
        hc = _upd_d(hc, ps, cnt, Wvc, bvc[None])
        pv = _segsum(hc, dst_p, src_p)
        hv = _upd_s(hv, pv, cnt, Wcv, bcv[None])

    bv3 = batch_vals.astype(jnp.int32).reshape(_GP, 1, _POOL_BLK)
    bc3 = batch_cons.astype(jnp.int32).reshape(_GP, 1, _POOL_BLK)
    return _pool(hv, hc, bv3, bc3, W_p1, b_p1[None], W_p2, b_p2[None])


# speedup vs baseline: 1.9193x; 1.7445x over previous
import functools

import jax
import jax.numpy as jnp
from jax import lax
from jax.experimental import pallas as pl
from jax.experimental.pallas import tpu as pltpu
from jax.experimental.pallas import tpu_sc as plsc

N = 10000
NPAD = 10240
HID = 128
E = 320000
B = 64
NC = 2
NS = 16
NW = NC * NS
CL = 128
SL = 120
CHUNKC = 80
EPADC = NW * CHUNKC * CL
GRP3 = 3
NGR = 28
EPAD = NW * NGR * GRP3 * SL
NACC = 10112
RPT = NACC // NS
RPTC = NPAD // NS

_mesh = plsc.VectorSubcoreMesh(core_axis_name="c", subcore_axis_name="s")


def _seg_body(h_hbm, gidx_hbm, sidx_hbm, out_hbm, gidx_v, sidx_v, rows, acc,
              g0, g1, g2, s0, s1, s2, isem):
    gsems = (g0, g1, g2)
    ssems = (s0, s1, s2)
    cid = lax.axis_index("c")
    sid = lax.axis_index("s")
    wid = sid * NC + cid

    def _zr(r, carry):
        for u in range(HID // 16):
            rows[0, r, pl.ds(u * 16, 16)] = jnp.zeros((16,), jnp.float32)
        return carry
    lax.fori_loop(0, SL, _zr, 0)
    base = sid * RPT
    for k in range(RPT // SL):
        pltpu.sync_copy(rows.at[0], acc.at[pl.ds(base + k * SL, SL)])
    if RPT % SL:
        pltpu.sync_copy(rows.at[0, pl.ds(0, RPT % SL)],
                        acc.at[pl.ds(base + (RPT // SL) * SL, RPT % SL)])
    plsc.subcore_barrier()

    pltpu.sync_copy(gidx_hbm.at[wid, 0], gidx_v.at[0])
    pltpu.sync_copy(sidx_hbm.at[wid, 0], sidx_v.at[0])
    for b in range(GRP3):
        pltpu.async_copy(h_hbm.at[gidx_v.at[0, b]], rows.at[b], gsems[b])

    def _grp(t, carry):
        islot = lax.rem(t, 2)

        @pl.when(t < NGR - 1)
        def _():
            pltpu.async_copy(gidx_hbm.at[wid, t + 1], gidx_v.at[1 - islot],
                             isem)
            pltpu.async_copy(sidx_hbm.at[wid, t + 1], sidx_v.at[1 - islot],
                             isem)

        for b in range(GRP3):
            pltpu.make_async_copy(h_hbm.at[gidx_v.at[islot, b]], rows.at[b],
                                  gsems[b]).wait()
            pltpu.async_copy(rows.at[b], acc.at[sidx_v.at[islot, b]],
                             ssems[b], add=True)

        @pl.when(t < NGR - 1)
        def _():
            pltpu.make_async_copy(gidx_hbm.at[wid, 0], gidx_v.at[1 - islot],
                                  isem).wait()
            pltpu.make_async_copy(sidx_hbm.at[wid, 0], sidx_v.at[1 - islot],
                                  isem).wait()

        for b in range(GRP3):
            pltpu.make_async_copy(rows.at[b], acc.at[sidx_v.at[islot, b]],
                                  ssems[b]).wait()

            @pl.when(t < NGR - 1)
            def _(b=b):
                pltpu.async_copy(h_hbm.at[gidx_v.at[1 - islot, b]],
                                 rows.at[b], gsems[b])
        return carry
    lax.fori_loop(0, NGR, _grp, 0)

    plsc.subcore_barrier()
    pltpu.sync_copy(acc.at[pl.ds(base, RPT)], out_hbm.at[cid, pl.ds(base, RPT)])


_segsum = functools.partial(
    pl.kernel,
    out_type=jax.ShapeDtypeStruct((NC, NPAD, HID), jnp.float32),
    mesh=_mesh,
    scratch_types=(
        [pltpu.VMEM((2, GRP3, SL), jnp.int32),
         pltpu.VMEM((2, GRP3, SL), jnp.int32),
         pltpu.VMEM((GRP3, SL, HID), jnp.float32),
         pltpu.VMEM_SHARED((NACC, HID), jnp.float32)]
        + [pltpu.SemaphoreType.DMA] * 7),
)(_seg_body)


def _cnt_body(sidx_hbm, didx_hbm, const_hbm, out_hbm, sidx_v, didx_v, ones_v,
              acc, sem):
    cid = lax.axis_index("c")
    sid = lax.axis_index("s")
    wid = sid * NC + cid
    pltpu.sync_copy(sidx_hbm.at[wid], sidx_v)
    pltpu.sync_copy(didx_hbm.at[wid], didx_v)

    pltpu.sync_copy(const_hbm.at[0], ones_v)
    base = sid * RPTC
    for k in range(RPTC // CL):
        pltpu.sync_copy(ones_v, acc.at[pl.ds(base + k * CL, CL)])
    plsc.subcore_barrier()

    def _pass(idx_v, plane):
        pltpu.sync_copy(const_hbm.at[plane], ones_v)

        FG = 4

        def _grp(g, carry):
            for b in range(FG):
                pltpu.async_copy(ones_v, acc.at[idx_v.at[g * FG + b]], sem,
                                 add=True)
            for b in range(FG):
                pltpu.make_async_copy(ones_v, acc.at[idx_v.at[g * FG + b]],
                                      sem).wait()
            return carry
        lax.fori_loop(0, CHUNKC // FG, _grp, 0)

    _pass(sidx_v, 1)
    _pass(didx_v, 2)

    plsc.subcore_barrier()
    pltpu.sync_copy(acc.at[pl.ds(base, RPTC)],
                    out_hbm.at[cid, pl.ds(base, RPTC)])


_counts = functools.partial(
    pl.kernel,
    out_type=jax.ShapeDtypeStruct((NC, NPAD, HID), jnp.float32),
    mesh=_mesh,
    scratch_types=[
        pltpu.VMEM((CHUNKC, CL), jnp.int32),
        pltpu.VMEM((CHUNKC, CL), jnp.int32),
        pltpu.VMEM((CL, HID), jnp.float32),
        pltpu.VMEM_SHARED((NPAD, HID), jnp.float32),
        pltpu.SemaphoreType.DMA,
    ],
)(_cnt_body)


def _enc_body(x_ref, w1_ref, b1_ref, w2_ref, b2_ref, o_ref):
    h = jnp.maximum(
        jnp.dot(x_ref[...], w1_ref[...], preferred_element_type=jnp.float32)
        + b1_ref[...], 0.0)
    o_ref[...] = jnp.maximum(
        jnp.dot(h, w2_ref[...], preferred_element_type=jnp.float32)
        + b2_ref[...], 0.0)


_ENC_BLK = 1024
_enc = pl.pallas_call(
    _enc_body,
    grid=(NPAD // _ENC_BLK,),
    in_specs=[
        pl.BlockSpec((_ENC_BLK, HID), lambda i: (i, 0)),
        pl.BlockSpec((HID, HID), lambda i: (0, 0)),
        pl.BlockSpec((1, HID), lambda i: (0, 0)),
        pl.BlockSpec((HID, HID), lambda i: (0, 0)),
        pl.BlockSpec((1, HID), lambda i: (0, 0)),
    ],
    out_specs=pl.BlockSpec((_ENC_BLK, HID), lambda i: (i, 0)),
    out_shape=jax.ShapeDtypeStruct((NPAD, HID), jnp.float32),
)


def _upd_body(col, h_ref, p_ref, c_ref, w_ref, b_ref, o_ref):
    s = p_ref[0] + p_ref[1]
    cnt = c_ref[0, :, col:col + 1] + c_ref[1, :, col:col + 1]
    m = s / jnp.maximum(cnt, 1.0)
    o_ref[...] = jnp.maximum(
        h_ref[...]
        + jnp.dot(m, w_ref[...], preferred_element_type=jnp.float32)
        + b_ref[...], 0.0)


_UPD_BLK = 1280


def _make_upd(col):
    return pl.pallas_call(
        functools.partial(_upd_body, col),
        grid=(NPAD // _UPD_BLK,),
        in_specs=[
            pl.BlockSpec((_UPD_BLK, HID), lambda i: (i, 0)),
            pl.BlockSpec((NC, _UPD_BLK, HID), lambda i: (0, i, 0)),
            pl.BlockSpec((NC, _UPD_BLK, HID), lambda i: (0, i, 0)),
            pl.BlockSpec((HID, HID), lambda i: (0, 0)),
            pl.BlockSpec((1, HID), lambda i: (0, 0)),
        ],
        out_specs=pl.BlockSpec((_UPD_BLK, HID), lambda i: (i, 0)),
        out_shape=jax.ShapeDtypeStruct((NPAD, HID), jnp.float32),
    )


_upd_s = _make_upd(0)
_upd_d = _make_upd(1)


_POOL_BLK = 1000
_GP = N // _POOL_BLK


def _pool_body(hv_ref, hc_ref, bv_ref, bc_ref, w1_ref, b1_ref, w2_ref, b2_ref,
               o_ref, pv_s, pc_s, cv_s, cc_s):
    i = pl.program_id(0)

    @pl.when(i == 0)
    def _():
        pv_s[...] = jnp.zeros((B, HID), jnp.float32)
        pc_s[...] = jnp.zeros((B, HID), jnp.float32)
        cv_s[...] = jnp.zeros((B, HID), jnp.float32)
        cc_s[...] = jnp.zeros((B, HID), jnp.float32)

    ids = lax.broadcasted_iota(jnp.int32, (B, _POOL_BLK), 0)
    ohv = (ids == bv_ref[0, 0, :][None, :]).astype(jnp.float32)
    ohc = (ids == bc_ref[0, 0, :][None, :]).astype(jnp.float32)
    pv_s[...] += jnp.dot(ohv, hv_ref[...], preferred_element_type=jnp.float32)
    pc_s[...] += jnp.dot(ohc, hc_ref[...], preferred_element_type=jnp.float32)
    cv_s[...] += jnp.broadcast_to(jnp.sum(ohv, axis=1)[:, None], (B, HID))
    cc_s[...] += jnp.broadcast_to(jnp.sum(ohc, axis=1)[:, None], (B, HID))

    @pl.when(i == _GP - 1)
    def _():
        mv = pv_s[...] / jnp.maximum(cv_s[...], 1.0)
        mc = pc_s[...] / jnp.maximum(cc_s[...], 1.0)
        emb = jnp.concatenate([mv, mc], axis=1)
        hh = jnp.maximum(
            jnp.dot(emb, w1_ref[...], preferred_element_type=jnp.float32)
            + b1_ref[...], 0.0)
        o_ref[...] = (jnp.dot(hh, w2_ref[...],
                              preferred_element_type=jnp.float32)
                      + b2_ref[...])


_pool = pl.pallas_call(
    _pool_body,
    grid=(_GP,),
    in_specs=[
        pl.BlockSpec((_POOL_BLK, HID), lambda i: (i, 0)),
        pl.BlockSpec((_POOL_BLK, HID), lambda i: (i, 0)),
        pl.BlockSpec((1, 1, _POOL_BLK), lambda i: (i, 0, 0)),
        pl.BlockSpec((1, 1, _POOL_BLK), lambda i: (i, 0, 0)),
        pl.BlockSpec((2 * HID, 4 * HID), lambda i: (0, 0)),
        pl.BlockSpec((1, 4 * HID), lambda i: (0, 0)),
        pl.BlockSpec((4 * HID, HID), lambda i: (0, 0)),
        pl.BlockSpec((1, HID), lambda i: (0, 0)),
    ],
    out_specs=pl.BlockSpec((B, HID), lambda i: (0, 0)),
    out_shape=jax.ShapeDtypeStruct((B, HID), jnp.float32),
    scratch_shapes=[
        pltpu.VMEM((B, HID), jnp.float32),
        pltpu.VMEM((B, HID), jnp.float32),
        pltpu.VMEM((B, HID), jnp.float32),
        pltpu.VMEM((B, HID), jnp.float32),
    ],
)


def kernel(x_vals, x_cons, edge_index, batch_vals, batch_cons,
           W_ev1, b_ev1, W_ev2, b_ev2, W_ec1, b_ec1, W_ec2, b_ec2,
           W_v2c_0, b_v2c_0, W_c2v_0, b_c2v_0, W_v2c_1, b_v2c_1,
           W_c2v_1, b_c2v_1, W_p1, b_p1, W_p2, b_p2):
    src = edge_index[0].astype(jnp.int32)
    dst = edge_index[1].astype(jnp.int32)
    pad = jnp.full((EPAD - E,), N, dtype=jnp.int32)
    padc = jnp.full((EPADC - E,), N, dtype=jnp.int32)
    src_p = jnp.concatenate([src, pad]).reshape(NW, NGR, GRP3, SL)
    dst_p = jnp.concatenate([dst, pad]).reshape(NW, NGR, GRP3, SL)
    src_pc = jnp.concatenate([src, padc]).reshape(NW, CHUNKC, CL)
    dst_pc = jnp.concatenate([dst, padc]).reshape(NW, CHUNKC, CL)
    xv = jnp.pad(x_vals, ((0, NPAD - N), (0, 0)))
    xc = jnp.pad(x_cons, ((0, NPAD - N), (0, 0)))

    onehot_rows = (jnp.zeros((3, CL, HID), jnp.float32)
                   .at[1, :, 0].set(1.0).at[2, :, 1].set(1.0))
    cnt = _counts(src_pc, dst_pc, onehot_rows)
    hv = _enc(xv, W_ev1, b_ev1[None], W_ev2, b_ev2[None])
    hc = _enc(xc, W_ec1, b_ec1[None], W_ec2, b_ec2[None])

    for Wvc, bvc, Wcv, bcv in ((W_v2c_0, b_v2c_0, W_c2v_0, b_c2v_0),
                               (W_v2c_1, b_v2c_1, W_c2v_1, b_c2v_1)):
        ps = _segsum(hv, src_p, dst_p)
        hc = _upd_d(hc, ps, cnt, Wvc, bvc[None])
        pv = _segsum(hc, dst_p, src_p)
        hv = _upd_s(hv, pv, cnt, Wcv, bcv[None])

    bv3 = batch_vals.astype(jnp.int32).reshape(_GP, 1, _POOL_BLK)
    bc3 = batch_cons.astype(jnp.int32).reshape(_GP, 1, _POOL_BLK)
    return _pool(hv, hc, bv3, bc3, W_p1, b_p1[None], W_p2, b_p2[None])
